# baseline (speedup 1.0000x reference)
import functools

import jax
import jax.numpy as jnp
from jax import lax
from jax.experimental import pallas as pl
from jax.experimental.pallas import tpu as pltpu
from jax.experimental.pallas import tpu_sc as plsc

N = 10000
E = 160000
NW = 32
B = 128
NB = 40
E_PAD = NW * NB * B
N_ACC = 10240
RPS = N_ACC // 16
NBA = 40
NBB = 40
FAST_CORE = 1

_MESH = plsc.VectorSubcoreMesh(core_axis_name="c", subcore_axis_name="s")


@functools.partial(
    pl.kernel, mesh=_MESH,
    out_type=jax.ShapeDtypeStruct((2, N_ACC, 128), jnp.float32),
    scratch_types=[
        pltpu.VMEM((NBA, B), jnp.int32),
        pltpu.VMEM((NBA, B), jnp.int32),
        pltpu.VMEM((B, 128), jnp.float32),
        pltpu.VMEM((B, 128), jnp.float32),
        pltpu.VMEM_SHARED((N_ACC, 128), jnp.float32),
        pltpu.SemaphoreType.DMA,
        pltpu.SemaphoreType.DMA,
    ],
)
def _agg(table_hbm, srca_hbm, dsta_hbm, srcb_hbm, dstb_hbm, zeros_hbm,
         out_hbm, src_v, dst_v, rows_a, rows_b, acc_sh, sga, sgb):
    c = lax.axis_index("c")
    s = lax.axis_index("s")
    pltpu.sync_copy(zeros_hbm, acc_sh.at[pl.ds(s * RPS, RPS)])

    @pl.when(c == FAST_CORE)
    def _():
        pltpu.sync_copy(srca_hbm.at[s], src_v.at[pl.ds(0, NBA)])
        pltpu.sync_copy(dsta_hbm.at[s], dst_v.at[pl.ds(0, NBA)])

    @pl.when(c != FAST_CORE)
    def _():
        pltpu.sync_copy(srcb_hbm.at[s], src_v.at[pl.ds(0, NBB)])
        pltpu.sync_copy(dstb_hbm.at[s], dst_v.at[pl.ds(0, NBB)])

    plsc.subcore_barrier()

    def run(nb):
        pltpu.async_copy(table_hbm.at[src_v.at[0]], rows_a, sga)

        def body(h, carry):
            j0 = 2 * h
            pltpu.async_copy(table_hbm.at[src_v.at[j0 + 1]], rows_b, sgb)
            pltpu.make_async_copy(
                table_hbm.at[src_v.at[0]], rows_a, sga).wait()
            pltpu.sync_copy(rows_a, acc_sh.at[dst_v.at[j0]], add=True)

            @pl.when(h < nb // 2 - 1)
            def _():
                pltpu.async_copy(table_hbm.at[src_v.at[j0 + 2]], rows_a, sga)

            pltpu.make_async_copy(
                table_hbm.at[src_v.at[0]], rows_b, sgb).wait()
            pltpu.sync_copy(rows_b, acc_sh.at[dst_v.at[j0 + 1]], add=True)
            return carry

        lax.fori_loop(0, nb // 2, body, 0)

    @pl.when(c == FAST_CORE)
    def _():
        run(NBA)

    @pl.when(c != FAST_CORE)
    def _():
        run(NBB)

    plsc.subcore_barrier()
    pltpu.sync_copy(acc_sh.at[pl.ds(s * RPS, RPS)],
                    out_hbm.at[c, pl.ds(s * RPS, RPS)])


@functools.partial(
    pl.kernel, mesh=_MESH,
    out_type=jax.ShapeDtypeStruct((2, N_ACC, 128), jnp.float32),
    scratch_types=[
        pltpu.VMEM((NB, B), jnp.int32),
        pltpu.VMEM((B, 128), jnp.float32),
        pltpu.VMEM_SHARED((N_ACC, 128), jnp.float32),
        pltpu.SemaphoreType.DMA,
    ],
)
def _counts(ones_hbm, dst_hbm, zeros_hbm, out_hbm, dst_v, ones_v, acc_sh,
            sem):
    c = lax.axis_index("c")
    s = lax.axis_index("s")
    wid = s * 2 + c
    pltpu.sync_copy(zeros_hbm, acc_sh.at[pl.ds(s * RPS, RPS)])
    pltpu.sync_copy(dst_hbm.at[wid], dst_v)
    pltpu.sync_copy(ones_hbm, ones_v)
    plsc.subcore_barrier()

    def fire(j, carry):
        pltpu.async_copy(ones_v, acc_sh.at[dst_v.at[j]], sem, add=True)
        return carry

    def drain(j, carry):
        pltpu.make_async_copy(ones_v, acc_sh.at[dst_v.at[0]], sem).wait()
        return carry

    lax.fori_loop(0, NB, fire, 0)
    lax.fori_loop(0, NB, drain, 0)
    plsc.subcore_barrier()
    pltpu.sync_copy(acc_sh.at[pl.ds(s * RPS, RPS)],
                    out_hbm.at[c, pl.ds(s * RPS, RPS)])


def _dense(parts, cnt, x, Wl, b, Wr, relu):
    NP, D = x.shape
    F = Wl.shape[1]
    BN = 512

    def body(p_ref, c_ref, x_ref, wl_ref, b_ref, wr_ref, o_ref):
        cnt_b = jnp.maximum(c_ref[0] + c_ref[1], 1.0)
        mean = (p_ref[0] + p_ref[1]) / cnt_b
        acc = jnp.dot(mean, wl_ref[...], preferred_element_type=jnp.float32)
        acc = acc + jnp.dot(x_ref[...], wr_ref[...],
                            preferred_element_type=jnp.float32)
        acc = acc + b_ref[...]
        if relu:
            acc = jnp.maximum(acc, 0.0)
        o_ref[...] = acc

    return pl.pallas_call(
        body,
        grid=(NP // BN,),
        in_specs=[
            pl.BlockSpec((2, BN, D), lambda i: (0, i, 0)),
            pl.BlockSpec((2, BN, 1), lambda i: (0, i, 0)),
            pl.BlockSpec((BN, D), lambda i: (i, 0)),
            pl.BlockSpec((D, F), lambda i: (0, 0)),
            pl.BlockSpec((1, F), lambda i: (0, 0)),
            pl.BlockSpec((D, F), lambda i: (0, 0)),
        ],
        out_specs=pl.BlockSpec((BN, F), lambda i: (i, 0)),
        out_shape=jax.ShapeDtypeStruct((NP, F), jnp.float32),
    )(parts, cnt, x, Wl, b, Wr)


def kernel(x, edge_index, W1l, b1l, W1r, W2l, b2l, W2r):
    src = edge_index[0]
    dst = edge_index[1]
    pad = E_PAD - E
    src_p = jnp.concatenate([src, jnp.zeros((pad,), jnp.int32)])
    dst_p = jnp.concatenate([dst, jnp.full((pad,), N, jnp.int32)])
    src3 = src_p.reshape(NW, NB, B)
    dst3 = dst_p.reshape(NW, NB, B)
    ea = 16 * NBA * B
    srca = src_p[:ea].reshape(16, NBA, B)
    dsta = dst_p[:ea].reshape(16, NBA, B)
    srcb = src_p[ea:].reshape(16, NBB, B)
    dstb = dst_p[ea:].reshape(16, NBB, B)
    zeros = jnp.zeros((RPS, 128), jnp.float32)
    ones = jnp.ones((B, 128), jnp.float32)

    cntp = _counts(ones, dst3, zeros)
    cnt = cntp[:, :, 0:1]

    def agg(table):
        return _agg(table, srca, dsta, srcb, dstb, zeros)

    p0 = agg(x[:, :128])
    p1 = agg(x[:, 128:])
    parts1 = jnp.concatenate([p0, p1], axis=2)

    x_pad = jnp.pad(x, ((0, N_ACC - N), (0, 0)))
    h = _dense(parts1, cnt, x_pad, W1l, b1l.reshape(1, -1), W1r, relu=True)

    p2 = [agg(h[:, k * 128:(k + 1) * 128]) for k in range(4)]
    parts2 = jnp.concatenate(p2, axis=2)
    out = _dense(parts2, cnt, h, W2l, b2l.reshape(1, -1), W2r, relu=False)
    return out[:N]


# speedup vs baseline: 1.1480x; 1.1480x over previous
import functools

import jax
import jax.numpy as jnp
from jax import lax
from jax.experimental import pallas as pl
from jax.experimental.pallas import tpu as pltpu
from jax.experimental.pallas import tpu_sc as plsc

N = 10000
E = 160000
NW = 32
B = 128
NB = 40
E_PAD = NW * NB * B
N_ACC = 10240
RPS = N_ACC // 16

_MESH = plsc.VectorSubcoreMesh(core_axis_name="c", subcore_axis_name="s")


@functools.partial(
    pl.kernel, mesh=_MESH,
    out_type=jax.ShapeDtypeStruct((2, N_ACC, 128), jnp.float32),
    scratch_types=[
        pltpu.VMEM((NB, B), jnp.int32),
        pltpu.VMEM((NB, B), jnp.int32),
        pltpu.VMEM((B, 128), jnp.float32),
        pltpu.VMEM((B, 128), jnp.float32),
        pltpu.VMEM_SHARED((N_ACC, 128), jnp.float32),
        pltpu.SemaphoreType.DMA,
        pltpu.SemaphoreType.DMA,
    ],
)
def _agg(table_hbm, src_hbm, dst_hbm, zeros_hbm, out_hbm,
         src_v, dst_v, rows_a, rows_b, acc_sh, sga, sgb):
    c = lax.axis_index("c")
    s = lax.axis_index("s")
    wid = s * 2 + c
    pltpu.sync_copy(zeros_hbm, acc_sh.at[pl.ds(s * RPS, RPS)])
    pltpu.sync_copy(src_hbm.at[wid], src_v)
    pltpu.sync_copy(dst_hbm.at[wid], dst_v)
    plsc.subcore_barrier()

    pltpu.async_copy(table_hbm.at[src_v.at[0]], rows_a, sga)

    def body(h, carry):
        j0 = 2 * h
        pltpu.async_copy(table_hbm.at[src_v.at[j0 + 1]], rows_b, sgb)
        pltpu.make_async_copy(table_hbm.at[src_v.at[0]], rows_a, sga).wait()
        pltpu.sync_copy(rows_a, acc_sh.at[dst_v.at[j0]], add=True)

        @pl.when(h < NB // 2 - 1)
        def _():
            pltpu.async_copy(table_hbm.at[src_v.at[j0 + 2]], rows_a, sga)

        pltpu.make_async_copy(table_hbm.at[src_v.at[0]], rows_b, sgb).wait()
        pltpu.sync_copy(rows_b, acc_sh.at[dst_v.at[j0 + 1]], add=True)
        return carry

    lax.fori_loop(0, NB // 2, body, 0)
    plsc.subcore_barrier()
    pltpu.sync_copy(acc_sh.at[pl.ds(s * RPS, RPS)],
                    out_hbm.at[c, pl.ds(s * RPS, RPS)])


@functools.partial(
    pl.kernel, mesh=_MESH,
    out_type=jax.ShapeDtypeStruct((2, N_ACC, 128), jnp.float32),
    scratch_types=[
        pltpu.VMEM((NB, B), jnp.int32),
        pltpu.VMEM((B, 128), jnp.float32),
        pltpu.VMEM_SHARED((N_ACC, 128), jnp.float32),
        pltpu.SemaphoreType.DMA,
    ],
)
def _counts(ones_hbm, dst_hbm, zeros_hbm, out_hbm, dst_v, ones_v, acc_sh,
            sem):
    c = lax.axis_index("c")
    s = lax.axis_index("s")
    wid = s * 2 + c
    pltpu.sync_copy(zeros_hbm, acc_sh.at[pl.ds(s * RPS, RPS)])
    pltpu.sync_copy(dst_hbm.at[wid], dst_v)
    pltpu.sync_copy(ones_hbm, ones_v)
    plsc.subcore_barrier()

    def fire(j, carry):
        pltpu.async_copy(ones_v, acc_sh.at[dst_v.at[j]], sem, add=True)
        return carry

    def drain(j, carry):
        pltpu.make_async_copy(ones_v, acc_sh.at[dst_v.at[0]], sem).wait()
        return carry

    lax.fori_loop(0, NB, fire, 0)
    lax.fori_loop(0, NB, drain, 0)
    plsc.subcore_barrier()
    pltpu.sync_copy(acc_sh.at[pl.ds(s * RPS, RPS)],
                    out_hbm.at[c, pl.ds(s * RPS, RPS)])


def _dense(parts, cnt, x, Wl, b, Wr, relu):
    NP, D = x.shape
    F = Wl.shape[1]
    BN = 512

    def body(p_ref, c_ref, x_ref, wl_ref, b_ref, wr_ref, o_ref):
        cnt_b = jnp.maximum(c_ref[0] + c_ref[1], 1.0)
        mean = (p_ref[0] + p_ref[1]) / cnt_b
        acc = jnp.dot(mean, wl_ref[...], preferred_element_type=jnp.float32)
        acc = acc + jnp.dot(x_ref[...], wr_ref[...],
                            preferred_element_type=jnp.float32)
        acc = acc + b_ref[...]
        if relu:
            acc = jnp.maximum(acc, 0.0)
        o_ref[...] = acc

    return pl.pallas_call(
        body,
        grid=(NP // BN,),
        in_specs=[
            pl.BlockSpec((2, BN, D), lambda i: (0, i, 0)),
            pl.BlockSpec((2, BN, 1), lambda i: (0, i, 0)),
            pl.BlockSpec((BN, D), lambda i: (i, 0)),
            pl.BlockSpec((D, F), lambda i: (0, 0)),
            pl.BlockSpec((1, F), lambda i: (0, 0)),
            pl.BlockSpec((D, F), lambda i: (0, 0)),
        ],
        out_specs=pl.BlockSpec((BN, F), lambda i: (i, 0)),
        out_shape=jax.ShapeDtypeStruct((NP, F), jnp.float32),
    )(parts, cnt, x, Wl, b, Wr)


def kernel(x, edge_index, W1l, b1l, W1r, W2l, b2l, W2r):
    src = edge_index[0]
    dst = edge_index[1]
    pad = E_PAD - E
    src_p = jnp.concatenate([src, jnp.zeros((pad,), jnp.int32)])
    dst_p = jnp.concatenate([dst, jnp.full((pad,), N, jnp.int32)])
    src3 = src_p.reshape(NW, NB, B)
    dst3 = dst_p.reshape(NW, NB, B)
    zeros = jnp.zeros((RPS, 128), jnp.float32)
    ones = jnp.ones((B, 128), jnp.float32)

    cntp = _counts(ones, dst3, zeros)
    cnt = cntp[:, :, 0:1]

    def agg(table):
        return _agg(table, src3, dst3, zeros)

    p0 = agg(x[:, :128])
    p1 = agg(x[:, 128:])
    parts1 = jnp.concatenate([p0, p1], axis=2)

    x_pad = jnp.pad(x, ((0, N_ACC - N), (0, 0)))
    h = _dense(parts1, cnt, x_pad, W1l, b1l.reshape(1, -1), W1r, relu=True)

    p2 = [agg(h[:, k * 128:(k + 1) * 128]) for k in range(4)]
    parts2 = jnp.concatenate(p2, axis=2)
    out = _dense(parts2, cnt, h, W2l, b2l.reshape(1, -1), W2r, relu=False)
    return out[:N]
